# baseline (speedup 1.0000x reference)
import functools

import jax
import jax.numpy as jnp
from jax import lax
from jax.experimental import pallas as pl
from jax.experimental.pallas import tpu as pltpu
from jax.experimental.pallas import tpu_sc as plsc

B = 16384
F = 26
V = 100000
D = 16
FD = F * D
H0, H1 = 400, 400

NC, NS = 2, 16
NW = NC * NS
BF = B * F
CW = 128
CH = BF // NW // CW
CPF = B // CW
GRP = 4
NGROUPS = CH // GRP
GROWS = GRP * CW


def _sc_gather_body(table_hbm, idx_hbm, out_hbm, idx_v, buf0, buf1,
                    gsem0, gsem1):
    wid = lax.axis_index("s") * NC + lax.axis_index("c")
    pltpu.sync_copy(idx_hbm.at[wid], idx_v)

    bufs = (buf0, buf1)
    gsems = (gsem0, gsem1)

    def fire_group(gi, s):
        c0 = wid * CH + gi * GRP
        f = c0 // CPF
        for cc in range(GRP):
            j = gi * GRP + cc
            pltpu.async_copy(table_hbm.at[f].at[idx_v.at[j]],
                             bufs[s].at[pl.ds(cc * CW, CW)], gsems[s])

    fire_group(0, 0)

    def outer(k, _):
        for s in range(2):
            gi = 2 * k + s

            @pl.when(gi + 1 < NGROUPS)
            def _():
                fire_group(gi + 1, (s + 1) % 2)

            c0 = wid * CH + gi * GRP
            f = c0 // CPF
            b0 = (c0 % CPF) * CW
            for cc in range(GRP):
                j = gi * GRP + cc
                pltpu.make_async_copy(table_hbm.at[f].at[idx_v.at[j]],
                                      bufs[s].at[pl.ds(cc * CW, CW)],
                                      gsems[s]).wait()
            pltpu.sync_copy(bufs[s],
                            out_hbm.at[pl.ds(b0, GROWS), pl.ds(f * D, D)])
        return _

    lax.fori_loop(0, NGROUPS // 2, outer, None)


def _sc_gather(embed, idx3d):
    mesh = plsc.VectorSubcoreMesh(core_axis_name="c", subcore_axis_name="s")
    k = functools.partial(
        pl.kernel, mesh=mesh,
        out_type=jax.ShapeDtypeStruct((B, FD), jnp.float32),
        compiler_params=pltpu.CompilerParams(use_tc_tiling_on_sc=False),
        scratch_types=[
            pltpu.VMEM((CH, CW), jnp.int32),
            pltpu.VMEM((GROWS, D), jnp.float32),
            pltpu.VMEM((GROWS, D), jnp.float32),
            pltpu.SemaphoreType.DMA,
            pltpu.SemaphoreType.DMA,
        ],
    )(_sc_gather_body)
    return k(embed, idx3d)


TBL = 2048


def _tpose_body(in_ref, out_ref):
    out_ref[0] = in_ref[0].T


def _tpose(embT):
    nvb = (V + TBL - 1) // TBL
    return pl.pallas_call(
        _tpose_body,
        grid=(F, nvb),
        in_specs=[pl.BlockSpec((1, D, TBL), lambda f, j: (f, 0, j))],
        out_specs=pl.BlockSpec((1, TBL, D), lambda f, j: (f, j, 0)),
        out_shape=jax.ShapeDtypeStruct((F, V, D), jnp.float32),
    )(embT)


def _mlp_body(x_ref, w0_ref, b0_ref, w1_ref, b1_ref, w2_ref, b2_ref, o_ref):
    x = x_ref[...]
    h = jnp.dot(x, w0_ref[...], preferred_element_type=jnp.float32)
    h = jnp.maximum(h + b0_ref[...], 0.0)
    h = jnp.dot(h, w1_ref[...], preferred_element_type=jnp.float32)
    h = jnp.maximum(h + b1_ref[...], 0.0)
    l = jnp.dot(h, w2_ref[...], preferred_element_type=jnp.float32)
    l = l + b2_ref[...]
    o_ref[...] = jax.nn.sigmoid(l)


MB = 2048


def _mlp(xw, w0, b0, w1, b1, w2, b2):
    return pl.pallas_call(
        _mlp_body,
        grid=(B // MB,),
        in_specs=[
            pl.BlockSpec((MB, FD), lambda i: (i, 0)),
            pl.BlockSpec((FD, H0), lambda i: (0, 0)),
            pl.BlockSpec((1, H0), lambda i: (0, 0)),
            pl.BlockSpec((H0, H1), lambda i: (0, 0)),
            pl.BlockSpec((1, H1), lambda i: (0, 0)),
            pl.BlockSpec((H1, 1), lambda i: (0, 0)),
            pl.BlockSpec((1, 1), lambda i: (0, 0)),
        ],
        out_specs=pl.BlockSpec((MB, 1), lambda i: (i, 0)),
        out_shape=jax.ShapeDtypeStruct((B, 1), jnp.float32),
    )(xw, w0, b0.reshape(1, H0), w1, b1.reshape(1, H1), w2,
      b2.reshape(1, 1))


def kernel(indices, embed, w0, b0, w1, b1, w2, b2):
    idx3d = indices.T.reshape(NW, CH, CW).astype(jnp.int32)
    embT = jnp.transpose(embed, (0, 2, 1))
    table = _tpose(embT)
    xw = _sc_gather(table, idx3d)
    out = _mlp(xw, w0, b0, w1, b1, w2, b2)
    return out[:, 0]


# speedup vs baseline: 5.1224x; 5.1224x over previous
import functools

import jax
import jax.numpy as jnp
from jax import lax
from jax.experimental import pallas as pl
from jax.experimental.pallas import tpu as pltpu
from jax.experimental.pallas import tpu_sc as plsc

B = 16384
F = 26
V = 100000
D = 16
FD = F * D
H0, H1 = 400, 400

NC, NS = 2, 16
NW = NC * NS
BF = B * F
CW = 128
CH = BF // NW // CW
CPF = B // CW
GRP = 4
NGROUPS = CH // GRP
GROWS = GRP * CW


def _sc_gather_body(table_hbm, idx_hbm, out_hbm, idx_v, buf0, buf1,
                    gsem0, gsem1):
    wid = lax.axis_index("s") * NC + lax.axis_index("c")
    pltpu.sync_copy(idx_hbm.at[wid], idx_v)

    bufs = (buf0, buf1)
    gsems = (gsem0, gsem1)

    def fire_group(gi, s):
        c0 = wid * CH + gi * GRP
        f = c0 // CPF
        for cc in range(GRP):
            j = gi * GRP + cc
            pltpu.async_copy(table_hbm.at[f].at[idx_v.at[j]],
                             bufs[s].at[pl.ds(cc * CW, CW)], gsems[s])

    fire_group(0, 0)

    def outer(k, _):
        for s in range(2):
            gi = 2 * k + s

            @pl.when(gi + 1 < NGROUPS)
            def _():
                fire_group(gi + 1, (s + 1) % 2)

            c0 = wid * CH + gi * GRP
            f = c0 // CPF
            b0 = (c0 % CPF) * CW
            for cc in range(GRP):
                j = gi * GRP + cc
                pltpu.make_async_copy(table_hbm.at[f].at[idx_v.at[j]],
                                      bufs[s].at[pl.ds(cc * CW, CW)],
                                      gsems[s]).wait()
            pltpu.sync_copy(bufs[s],
                            out_hbm.at[pl.ds(b0, GROWS), pl.ds(f * D, D)])
        return _

    lax.fori_loop(0, NGROUPS // 2, outer, None)


def _sc_gather(embed, idx3d):
    mesh = plsc.VectorSubcoreMesh(core_axis_name="c", subcore_axis_name="s")
    k = functools.partial(
        pl.kernel, mesh=mesh,
        out_type=jax.ShapeDtypeStruct((B, FD), jnp.float32),
        compiler_params=pltpu.CompilerParams(use_tc_tiling_on_sc=False),
        scratch_types=[
            pltpu.VMEM((CH, CW), jnp.int32),
            pltpu.VMEM((GROWS, D), jnp.float32),
            pltpu.VMEM((GROWS, D), jnp.float32),
            pltpu.SemaphoreType.DMA,
            pltpu.SemaphoreType.DMA,
        ],
    )(_sc_gather_body)
    return k(embed, idx3d)


TBL = 8192
TROWS = TBL * D // 128
VROWS = 13312
VP = VROWS * 128 // D


def _tpose_body(in_ref, out_ref):
    x = in_ref[0]
    w = jnp.concatenate(
        [x[:, TROWS * s:TROWS * (s + 1)] for s in range(8)], axis=0)
    out_ref[0] = w.T


def _tpose(embT):
    return pl.pallas_call(
        _tpose_body,
        grid=(F, VROWS // TROWS),
        in_specs=[pl.BlockSpec((1, D, TBL), lambda f, j: (f, 0, j))],
        out_specs=pl.BlockSpec((1, TROWS, 128), lambda f, j: (f, j, 0)),
        out_shape=jax.ShapeDtypeStruct((F, VROWS, 128), jnp.float32),
    )(embT)


def _mlp_body(x_ref, w0_ref, b0_ref, w1_ref, b1_ref, w2_ref, b2_ref, o_ref):
    x = x_ref[...]
    h = jnp.dot(x, w0_ref[...], preferred_element_type=jnp.float32)
    h = jnp.maximum(h + b0_ref[...], 0.0)
    h = jnp.dot(h, w1_ref[...], preferred_element_type=jnp.float32)
    h = jnp.maximum(h + b1_ref[...], 0.0)
    l = jnp.dot(h, w2_ref[...], preferred_element_type=jnp.float32)
    l = l + b2_ref[...]
    o_ref[...] = jax.nn.sigmoid(l)


MB = 2048


def _mlp(xw, w0, b0, w1, b1, w2, b2):
    return pl.pallas_call(
        _mlp_body,
        grid=(B // MB,),
        in_specs=[
            pl.BlockSpec((MB, FD), lambda i: (i, 0)),
            pl.BlockSpec((FD, H0), lambda i: (0, 0)),
            pl.BlockSpec((1, H0), lambda i: (0, 0)),
            pl.BlockSpec((H0, H1), lambda i: (0, 0)),
            pl.BlockSpec((1, H1), lambda i: (0, 0)),
            pl.BlockSpec((H1, 1), lambda i: (0, 0)),
            pl.BlockSpec((1, 1), lambda i: (0, 0)),
        ],
        out_specs=pl.BlockSpec((MB, 1), lambda i: (i, 0)),
        out_shape=jax.ShapeDtypeStruct((B, 1), jnp.float32),
    )(xw, w0, b0.reshape(1, H0), w1, b1.reshape(1, H1), w2,
      b2.reshape(1, 1))


def kernel(indices, embed, w0, b0, w1, b1, w2, b2):
    v = indices.astype(jnp.int32)
    rows = (v >> 13) * 8192 + (v & 1023) * 8 + ((v >> 10) & 7)
    idx3d = rows.T.reshape(NW, CH, CW)
    embT = jnp.transpose(embed, (0, 2, 1))
    table = _tpose(embT).reshape(F, VP, D)
    xw = _sc_gather(table, idx3d)
    out = _mlp(xw, w0, b0, w1, b1, w2, b2)
    return out[:, 0]


# speedup vs baseline: 6.1275x; 1.1962x over previous
import functools

import jax
import jax.numpy as jnp
from jax import lax
from jax.experimental import pallas as pl
from jax.experimental.pallas import tpu as pltpu
from jax.experimental.pallas import tpu_sc as plsc

B = 16384
F = 26
V = 100000
D = 16
FD = F * D
H0, H1 = 400, 400

NC, NS = 2, 16
NW = NC * NS
BF = B * F
CW = 128
CH = BF // NW // CW
CPF = B // CW
GRP = 4
NGROUPS = CH // GRP
GROWS = GRP * CW


def _sc_gather_body(table_hbm, idx_hbm, out_hbm, idx_v, buf0, buf1,
                    gsem0, gsem1):
    wid = lax.axis_index("s") * NC + lax.axis_index("c")
    pltpu.sync_copy(idx_hbm.at[wid], idx_v)

    bufs = (buf0, buf1)
    gsems = (gsem0, gsem1)

    def fire_group(gi, s):
        c0 = wid * CH + gi * GRP
        f = c0 // CPF
        for cc in range(GRP):
            j = gi * GRP + cc
            pltpu.async_copy(table_hbm.at[f].at[idx_v.at[j]],
                             bufs[s].at[pl.ds(cc * CW, CW)], gsems[s])

    fire_group(0, 0)

    def outer(k, _):
        for s in range(2):
            gi = 2 * k + s

            @pl.when(gi + 1 < NGROUPS)
            def _():
                fire_group(gi + 1, (s + 1) % 2)

            c0 = wid * CH + gi * GRP
            f = c0 // CPF
            b0 = (c0 % CPF) * CW
            for cc in range(GRP):
                j = gi * GRP + cc
                pltpu.make_async_copy(table_hbm.at[f].at[idx_v.at[j]],
                                      bufs[s].at[pl.ds(cc * CW, CW)],
                                      gsems[s]).wait()
            pltpu.sync_copy(bufs[s],
                            out_hbm.at[pl.ds(b0, GROWS), pl.ds(f * D, D)])
        return _

    lax.fori_loop(0, NGROUPS // 2, outer, None)


def _sc_gather(embed, idx3d):
    mesh = plsc.VectorSubcoreMesh(core_axis_name="c", subcore_axis_name="s")
    k = functools.partial(
        pl.kernel, mesh=mesh,
        out_type=jax.ShapeDtypeStruct((B, FD), jnp.float32),
        compiler_params=pltpu.CompilerParams(use_tc_tiling_on_sc=False),
        scratch_types=[
            pltpu.VMEM((CH, CW), jnp.int32),
            pltpu.VMEM((GROWS, D), jnp.float32),
            pltpu.VMEM((GROWS, D), jnp.float32),
            pltpu.SemaphoreType.DMA,
            pltpu.SemaphoreType.DMA,
        ],
    )(_sc_gather_body)
    return k(embed, idx3d)


TBL = 16384
TROWS = TBL * D // 128
NVB = (V + TBL - 1) // TBL
VROWS = NVB * TROWS
VP = VROWS * 128 // D


def _tpose_body(in_ref, out_ref):
    x = in_ref[0]
    w = jnp.concatenate(
        [x[:, TROWS * s:TROWS * (s + 1)] for s in range(8)], axis=0)
    out_ref[0] = w.T


def _tpose(embT):
    return pl.pallas_call(
        _tpose_body,
        grid=(F, NVB),
        in_specs=[pl.BlockSpec((1, D, TBL), lambda f, j: (f, 0, j))],
        out_specs=pl.BlockSpec((1, TROWS, 128), lambda f, j: (f, j, 0)),
        out_shape=jax.ShapeDtypeStruct((F, VROWS, 128), jnp.float32),
    )(embT)


def _mlp_body(x_ref, w0_ref, b0_ref, w1_ref, b1_ref, w2_ref, b2_ref, o_ref):
    x = x_ref[...]
    h = jnp.dot(x, w0_ref[...], preferred_element_type=jnp.float32)
    h = jnp.maximum(h + b0_ref[...], 0.0)
    h = jnp.dot(h, w1_ref[...], preferred_element_type=jnp.float32)
    h = jnp.maximum(h + b1_ref[...], 0.0)
    l = jnp.dot(h, w2_ref[...], preferred_element_type=jnp.float32)
    l = l + b2_ref[...]
    o_ref[...] = jax.nn.sigmoid(l)


MB = 2048


def _mlp(xw, w0, b0, w1, b1, w2, b2):
    return pl.pallas_call(
        _mlp_body,
        grid=(B // MB,),
        in_specs=[
            pl.BlockSpec((MB, FD), lambda i: (i, 0)),
            pl.BlockSpec((FD, H0), lambda i: (0, 0)),
            pl.BlockSpec((1, H0), lambda i: (0, 0)),
            pl.BlockSpec((H0, H1), lambda i: (0, 0)),
            pl.BlockSpec((1, H1), lambda i: (0, 0)),
            pl.BlockSpec((H1, 1), lambda i: (0, 0)),
            pl.BlockSpec((1, 1), lambda i: (0, 0)),
        ],
        out_specs=pl.BlockSpec((MB, 1), lambda i: (i, 0)),
        out_shape=jax.ShapeDtypeStruct((B, 1), jnp.float32),
    )(xw, w0, b0.reshape(1, H0), w1, b1.reshape(1, H1), w2,
      b2.reshape(1, 1))


def kernel(indices, embed, w0, b0, w1, b1, w2, b2):
    v = indices.astype(jnp.int32)
    rows = ((v // TBL) * TBL + (v % TROWS) * 8 + (v // TROWS) % 8)
    idx3d = rows.T.reshape(NW, CH, CW)
    embT = jnp.transpose(embed, (0, 2, 1))
    table = _tpose(embT).reshape(F, VP, D)
    xw = _sc_gather(table, idx3d)
    out = _mlp(xw, w0, b0, w1, b1, w2, b2)
    return out[:, 0]


# speedup vs baseline: 6.9296x; 1.1309x over previous
import functools

import jax
import jax.numpy as jnp
from jax import lax
from jax.experimental import pallas as pl
from jax.experimental.pallas import tpu as pltpu
from jax.experimental.pallas import tpu_sc as plsc

B = 16384
F = 26
V = 100000
D = 16
FD = F * D
H0, H1 = 400, 400

NC, NS = 2, 16
NW = NC * NS
BF = B * F
CW = 128
CH = BF // NW // CW
CPF = B // CW
GRP = 4
NGROUPS = CH // GRP
GROWS = GRP * CW


def _sc_gather_body(table_hbm, idx_hbm, out_hbm, idx_v, buf0, buf1,
                    gsem0, gsem1):
    wid = lax.axis_index("s") * NC + lax.axis_index("c")
    pltpu.sync_copy(idx_hbm.at[wid], idx_v)

    bufs = (buf0, buf1)
    gsems = (gsem0, gsem1)

    def fire_group(gi, s):
        c0 = wid * CH + gi * GRP
        f = c0 // CPF
        for cc in range(GRP):
            j = gi * GRP + cc
            pltpu.async_copy(table_hbm.at[f].at[idx_v.at[j]],
                             bufs[s].at[pl.ds(cc * CW, CW)], gsems[s])

    fire_group(0, 0)

    def outer(k, _):
        for s in range(2):
            gi = 2 * k + s

            @pl.when(gi + 1 < NGROUPS)
            def _():
                fire_group(gi + 1, (s + 1) % 2)

            c0 = wid * CH + gi * GRP
            f = c0 // CPF
            b0 = (c0 % CPF) * CW
            for cc in range(GRP):
                j = gi * GRP + cc
                pltpu.make_async_copy(table_hbm.at[f].at[idx_v.at[j]],
                                      bufs[s].at[pl.ds(cc * CW, CW)],
                                      gsems[s]).wait()
            pltpu.sync_copy(bufs[s],
                            out_hbm.at[pl.ds(b0, GROWS), pl.ds(f * D, D)])
        return _

    lax.fori_loop(0, NGROUPS // 2, outer, None)


def _sc_gather(embed, idx3d):
    mesh = plsc.VectorSubcoreMesh(core_axis_name="c", subcore_axis_name="s")
    k = functools.partial(
        pl.kernel, mesh=mesh,
        out_type=jax.ShapeDtypeStruct((B, FD), jnp.float32),
        compiler_params=pltpu.CompilerParams(use_tc_tiling_on_sc=False),
        scratch_types=[
            pltpu.VMEM((CH, CW), jnp.int32),
            pltpu.VMEM((GROWS, D), jnp.float32),
            pltpu.VMEM((GROWS, D), jnp.float32),
            pltpu.SemaphoreType.DMA,
            pltpu.SemaphoreType.DMA,
        ],
    )(_sc_gather_body)
    return k(embed, idx3d)


TBL = 20480
TROWS = TBL * D // 128
NVB = (V + TBL - 1) // TBL
VROWS = NVB * TROWS
VP = VROWS * 128 // D


def _tpose_body(in_ref, out_ref):
    x = in_ref[0]
    w = jnp.concatenate(
        [x[:, TROWS * s:TROWS * (s + 1)] for s in range(8)], axis=0)
    out_ref[0] = w.T


def _tpose(embT):
    return pl.pallas_call(
        _tpose_body,
        grid=(F, NVB),
        in_specs=[pl.BlockSpec((1, D, TBL), lambda f, j: (f, 0, j))],
        out_specs=pl.BlockSpec((1, TROWS, 128), lambda f, j: (f, j, 0)),
        out_shape=jax.ShapeDtypeStruct((F, VROWS, 128), jnp.float32),
    )(embT)


def _mlp_body(x_ref, w0_ref, b0_ref, w1_ref, b1_ref, w2_ref, b2_ref, o_ref):
    x = x_ref[...]
    h = jnp.dot(x, w0_ref[...], preferred_element_type=jnp.float32)
    h = jnp.maximum(h + b0_ref[...], 0.0)
    h = jnp.dot(h, w1_ref[...], preferred_element_type=jnp.float32)
    h = jnp.maximum(h + b1_ref[...], 0.0)
    l = jnp.dot(h, w2_ref[...], preferred_element_type=jnp.float32)
    l = l + b2_ref[...]
    o_ref[...] = jax.nn.sigmoid(l)


MB = 2048


def _mlp(xw, w0, b0, w1, b1, w2, b2):
    return pl.pallas_call(
        _mlp_body,
        grid=(B // MB,),
        in_specs=[
            pl.BlockSpec((MB, FD), lambda i: (i, 0)),
            pl.BlockSpec((FD, H0), lambda i: (0, 0)),
            pl.BlockSpec((1, H0), lambda i: (0, 0)),
            pl.BlockSpec((H0, H1), lambda i: (0, 0)),
            pl.BlockSpec((1, H1), lambda i: (0, 0)),
            pl.BlockSpec((H1, 1), lambda i: (0, 0)),
            pl.BlockSpec((1, 1), lambda i: (0, 0)),
        ],
        out_specs=pl.BlockSpec((MB, 1), lambda i: (i, 0)),
        out_shape=jax.ShapeDtypeStruct((B, 1), jnp.float32),
    )(xw, w0, b0.reshape(1, H0), w1, b1.reshape(1, H1), w2,
      b2.reshape(1, 1))


def kernel(indices, embed, w0, b0, w1, b1, w2, b2):
    v = indices.astype(jnp.int32)
    rows = ((v // TBL) * TBL + (v % TROWS) * 8 + (v // TROWS) % 8)
    idx3d = rows.T.reshape(NW, CH, CW)
    embT = jnp.transpose(embed, (0, 2, 1))
    table = _tpose(embT).reshape(F, VP, D)
    xw = _sc_gather(table, idx3d)
    out = _mlp(xw, w0, b0, w1, b1, w2, b2)
    return out[:, 0]


# speedup vs baseline: 8.4792x; 1.2236x over previous
import functools

import jax
import jax.numpy as jnp
from jax import lax
from jax.experimental import pallas as pl
from jax.experimental.pallas import tpu as pltpu
from jax.experimental.pallas import tpu_sc as plsc

B = 16384
F = 26
V = 100000
D = 16
FD = F * D
H0, H1 = 400, 400

NC, NS = 2, 16
NW = NC * NS
BF = B * F
CW = 128
CH = BF // NW // CW
CPF = B // CW
GRP = 4
NGROUPS = CH // GRP
GROWS = GRP * CW


def _sc_gather_body(table_hbm, idx_hbm, out_hbm, idx_v, buf0, buf1,
                    gsem0, gsem1):
    wid = lax.axis_index("s") * NC + lax.axis_index("c")
    pltpu.sync_copy(idx_hbm.at[wid], idx_v)

    bufs = (buf0, buf1)
    gsems = (gsem0, gsem1)

    def fire_group(gi, s):
        c0 = wid * CH + gi * GRP
        f = c0 // CPF
        for cc in range(GRP):
            j = gi * GRP + cc
            pltpu.async_copy(table_hbm.at[f].at[idx_v.at[j]],
                             bufs[s].at[pl.ds(cc * CW, CW)], gsems[s])

    fire_group(0, 0)

    def outer(k, _):
        for s in range(2):
            gi = 2 * k + s

            @pl.when(gi + 1 < NGROUPS)
            def _():
                fire_group(gi + 1, (s + 1) % 2)

            c0 = wid * CH + gi * GRP
            f = c0 // CPF
            b0 = (c0 % CPF) * CW
            for cc in range(GRP):
                j = gi * GRP + cc
                pltpu.make_async_copy(table_hbm.at[f].at[idx_v.at[j]],
                                      bufs[s].at[pl.ds(cc * CW, CW)],
                                      gsems[s]).wait()
            pltpu.sync_copy(bufs[s],
                            out_hbm.at[pl.ds(b0, GROWS), pl.ds(f * D, D)])
        return _

    lax.fori_loop(0, NGROUPS // 2, outer, None)


def _sc_gather(embed, idx3d):
    mesh = plsc.VectorSubcoreMesh(core_axis_name="c", subcore_axis_name="s")
    k = functools.partial(
        pl.kernel, mesh=mesh,
        out_type=jax.ShapeDtypeStruct((B, FD), jnp.float32),
        compiler_params=pltpu.CompilerParams(use_tc_tiling_on_sc=False),
        scratch_types=[
            pltpu.VMEM((CH, CW), jnp.int32),
            pltpu.VMEM((GROWS, D), jnp.float32),
            pltpu.VMEM((GROWS, D), jnp.float32),
            pltpu.SemaphoreType.DMA,
            pltpu.SemaphoreType.DMA,
        ],
    )(_sc_gather_body)
    return k(embed, idx3d)


TBL = 102400
TROWS = TBL * D // 128
NVB = (V + TBL - 1) // TBL
VROWS = NVB * TROWS
VP = VROWS * 128 // D


def _tpose_body(in_ref, out_ref):
    x = in_ref[0]
    w = jnp.concatenate(
        [x[:, TROWS * s:TROWS * (s + 1)] for s in range(8)], axis=0)
    out_ref[0] = w.T


def _tpose(embT):
    return pl.pallas_call(
        _tpose_body,
        grid=(F, NVB),
        in_specs=[pl.BlockSpec((1, D, TBL), lambda f, j: (f, 0, j))],
        out_specs=pl.BlockSpec((1, TROWS, 128), lambda f, j: (f, j, 0)),
        out_shape=jax.ShapeDtypeStruct((F, VROWS, 128), jnp.float32),
    )(embT)


def _mlp_body(x_ref, w0_ref, b0_ref, w1_ref, b1_ref, w2_ref, b2_ref, o_ref):
    x = x_ref[...]
    h = jnp.dot(x, w0_ref[...], preferred_element_type=jnp.float32)
    h = jnp.maximum(h + b0_ref[...], 0.0)
    h = jnp.dot(h, w1_ref[...], preferred_element_type=jnp.float32)
    h = jnp.maximum(h + b1_ref[...], 0.0)
    l = jnp.dot(h, w2_ref[...], preferred_element_type=jnp.float32)
    l = l + b2_ref[...]
    o_ref[...] = jax.nn.sigmoid(l)


MB = 2048


def _mlp(xw, w0, b0, w1, b1, w2, b2):
    return pl.pallas_call(
        _mlp_body,
        grid=(B // MB,),
        in_specs=[
            pl.BlockSpec((MB, FD), lambda i: (i, 0)),
            pl.BlockSpec((FD, H0), lambda i: (0, 0)),
            pl.BlockSpec((1, H0), lambda i: (0, 0)),
            pl.BlockSpec((H0, H1), lambda i: (0, 0)),
            pl.BlockSpec((1, H1), lambda i: (0, 0)),
            pl.BlockSpec((H1, 1), lambda i: (0, 0)),
            pl.BlockSpec((1, 1), lambda i: (0, 0)),
        ],
        out_specs=pl.BlockSpec((MB, 1), lambda i: (i, 0)),
        out_shape=jax.ShapeDtypeStruct((B, 1), jnp.float32),
    )(xw, w0, b0.reshape(1, H0), w1, b1.reshape(1, H1), w2,
      b2.reshape(1, 1))


def kernel(indices, embed, w0, b0, w1, b1, w2, b2):
    v = indices.astype(jnp.int32)
    rows = ((v // TBL) * TBL + (v % TROWS) * 8 + (v // TROWS) % 8)
    idx3d = rows.T.reshape(NW, CH, CW)
    embT = jnp.transpose(embed, (0, 2, 1))
    table = _tpose(embT).reshape(F, VP, D)
    xw = _sc_gather(table, idx3d)
    out = _mlp(xw, w0, b0, w1, b1, w2, b2)
    return out[:, 0]
